# baseline (speedup 1.0000x reference)
import functools

import jax
import jax.numpy as jnp
from jax import lax
from jax.experimental import pallas as pl
from jax.experimental.pallas import tpu as pltpu
from jax.experimental.pallas import tpu_sc as plsc

_B, _H, _W = 8, 384, 384
_NPIX = _H * _W
_NWORDS = _NPIX // 2
_TILES_PER_IMG = 4
_ROWS_PER_TILE = _H // _TILES_PER_IMG
_CHUNK_ROWS = 16
_CHUNK_PIX = _CHUNK_ROWS * _W
_NCHUNKS = _ROWS_PER_TILE // _CHUNK_ROWS
_GROUPS_PER_ROW = _W // 16
_PACK_CHUNKS = 24
_PACK_W = _NWORDS // _PACK_CHUNKS


def _sc_body(imgs_ref, dvf_ref, out_ref,
             img_v, lo_v, hi_v, dvf_v, out_v,
             lo_sem, hi_sem, dvf_sem, out_sem):
    cid = lax.axis_index("c")
    sid = lax.axis_index("s")
    wid = sid * 2 + cid
    b = wid // _TILES_PER_IMG
    q = wid % _TILES_PER_IMG
    row0 = q * _ROWS_PER_TILE
    ibase = b * _NPIX

    def _start_pack(s, slot):
        o = s * _PACK_W
        lo_d = pltpu.async_copy(
            imgs_ref.at[pl.ds(ibase + o, _PACK_W)], lo_v.at[slot], lo_sem)
        hi_d = pltpu.async_copy(
            imgs_ref.at[pl.ds(ibase + _NWORDS + o, _PACK_W)],
            hi_v.at[slot], hi_sem)
        return lo_d, hi_d

    pend = {0: _start_pack(0, 0)}
    for s in range(_PACK_CHUNKS):
        slot = s % 2
        if s + 1 < _PACK_CHUNKS:
            pend[s + 1] = _start_pack(s + 1, (s + 1) % 2)
        lo_d, hi_d = pend.pop(s)
        lo_d.wait()
        hi_d.wait()
        o = s * _PACK_W

        @plsc.parallel_loop(0, _PACK_W // 16, unroll=4)
        def _pack(g):
            lo = plsc.bitcast(lo_v[slot, pl.ds(g * 16, 16)], jnp.int32)
            hi = plsc.bitcast(hi_v[slot, pl.ds(g * 16, 16)], jnp.int32)
            lor = lax.shift_right_logical(lo + 0x8000, 16)
            hir = lax.shift_right_logical(hi + 0x8000, 16)
            img_v[pl.ds(o + g * 16, 16)] = lor | (hir << 16)

    lane = lax.iota(jnp.int32, 16)
    lanef = lane.astype(jnp.float32)

    def _start_dvf(ch, slot):
        crow = row0 + ch * _CHUNK_ROWS
        dsrc = (b * _H + crow) * 2 * _W
        return pltpu.async_copy(
            dvf_ref.at[pl.ds(dsrc, _CHUNK_PIX * 2)], dvf_v.at[slot], dvf_sem)

    dvf_pend = {0: _start_dvf(0, 0)}
    out_pend = {}
    for ch in range(_NCHUNKS):
        slot = ch % 2
        if ch + 1 < _NCHUNKS:
            dvf_pend[ch + 1] = _start_dvf(ch + 1, (ch + 1) % 2)
        dvf_pend.pop(ch).wait()
        if ch >= 2:
            out_pend.pop(ch - 2).wait()
        crow = row0 + ch * _CHUNK_ROWS

        def _row(r, carry):
            rowf = (crow + r).astype(jnp.float32)

            @plsc.parallel_loop(0, _GROUPS_PER_ROW, unroll=6)
            def _grp(t):
                p0 = r * _W + t * 16
                doff = 2 * r * _W + t * 16
                dx = dvf_v[slot, pl.ds(doff, 16)]
                dy = dvf_v[slot, pl.ds(doff + _W, 16)]

                fx = (t * 16).astype(jnp.float32) + lanef + dx
                fy = rowf + dy
                x0 = fx.astype(jnp.int32)
                y0 = fy.astype(jnp.int32)
                x1 = x0 + 1
                y1 = y0 + 1
                x0 = jnp.clip(x0, 0, _W - 1)
                x1 = jnp.clip(x1, 0, _W - 1)
                y0 = jnp.clip(y0, 0, _H - 1)
                y1 = jnp.clip(y1, 0, _H - 1)

                pa = y0 * _W + x0
                pb = y1 * _W + x0
                m0 = y0 < (_H // 2)
                m1 = y1 < (_H // 2)
                off0 = jnp.where(m0, 0, _NWORDS)
                off1 = jnp.where(m1, 0, _NWORDS)
                sh0 = jnp.where(m0, 16, 0)
                sh1 = jnp.where(m1, 16, 0)

                wa = plsc.load_gather(img_v, [pa - off0])
                wb = plsc.load_gather(img_v, [pb - off1])
                wc = plsc.load_gather(img_v, [pa + 1 - off0])
                wd = plsc.load_gather(img_v, [pb + 1 - off1])
                edge = x1 == x0
                va = plsc.bitcast(wa << sh0, jnp.float32)
                vb = plsc.bitcast(wb << sh1, jnp.float32)
                vc = jnp.where(edge, va, plsc.bitcast(wc << sh0, jnp.float32))
                vd = jnp.where(edge, vb, plsc.bitcast(wd << sh1, jnp.float32))

                x0f = x0.astype(jnp.float32)
                x1f = x1.astype(jnp.float32)
                y0f = y0.astype(jnp.float32)
                y1f = y1.astype(jnp.float32)
                wx1 = x1f - fx
                wx0 = fx - x0f
                wy1 = y1f - fy
                wy0 = fy - y0f
                res = ((wx1 * wy1) * va + (wx1 * wy0) * vb
                       + (wx0 * wy1) * vc + (wx0 * wy0) * vd)
                out_v[slot, pl.ds(p0, 16)] = res

            return carry

        lax.fori_loop(0, _CHUNK_ROWS, _row, jnp.int32(0))

        dst = ibase + crow * _W
        out_pend[ch] = pltpu.async_copy(
            out_v.at[slot], out_ref.at[pl.ds(dst, _CHUNK_PIX)], out_sem)
    for ch in sorted(out_pend):
        out_pend[ch].wait()


@jax.jit
def _run(imgs_flat, dvf_flat):
    mesh = plsc.VectorSubcoreMesh(core_axis_name="c", subcore_axis_name="s")
    fn = pl.kernel(
        _sc_body,
        out_type=jax.ShapeDtypeStruct((_B * _NPIX,), jnp.float32),
        name="bilerp_sc",
        mesh=mesh,
        scratch_types=[
            pltpu.VMEM((_NWORDS,), jnp.int32),
            pltpu.VMEM((2, _PACK_W), jnp.float32),
            pltpu.VMEM((2, _PACK_W), jnp.float32),
            pltpu.VMEM((2, _CHUNK_PIX * 2), jnp.float32),
            pltpu.VMEM((2, _CHUNK_PIX), jnp.float32),
            pltpu.SemaphoreType.DMA,
            pltpu.SemaphoreType.DMA,
            pltpu.SemaphoreType.DMA,
            pltpu.SemaphoreType.DMA,
        ],
        compiler_params=pltpu.CompilerParams(needs_layout_passes=False),
    )
    return fn(imgs_flat, dvf_flat)


def kernel(imgs, dvfs):
    B, H, W, C = imgs.shape
    imgs_flat = imgs.reshape(-1)
    dvf_flat = jnp.transpose(dvfs, (0, 1, 3, 2)).reshape(-1)
    out = _run(imgs_flat, dvf_flat)
    return out.reshape(B, H, W, C)


# speedup vs baseline: 1.1293x; 1.1293x over previous
import functools

import jax
import jax.numpy as jnp
from jax import lax
from jax.experimental import pallas as pl
from jax.experimental.pallas import tpu as pltpu
from jax.experimental.pallas import tpu_sc as plsc

_B, _H, _W = 8, 384, 384
_NPIX = _H * _W
_NWORDS = _NPIX // 2
_TILES_PER_IMG = 4
_ROWS_PER_TILE = _H // _TILES_PER_IMG
_CHUNK_ROWS = 16
_CHUNK_PIX = _CHUNK_ROWS * _W
_NCHUNKS = _ROWS_PER_TILE // _CHUNK_ROWS
_GROUPS_PER_ROW = _W // 16
_PACK_CHUNKS = 16
_PACK_W = _NWORDS // _PACK_CHUNKS


def _sc_body(imgs_ref, dvf_ref, out_ref,
             img_v, lo_v, hi_v, dvf_v, out_v,
             lo_sem, hi_sem, dvf_sem, out_sem):
    cid = lax.axis_index("c")
    sid = lax.axis_index("s")
    wid = sid * 2 + cid
    b = wid // _TILES_PER_IMG
    q = wid % _TILES_PER_IMG
    row0 = q * _ROWS_PER_TILE
    ibase = b * _NPIX

    def _start_pack(s, slot):
        o = s * _PACK_W
        lo_d = pltpu.async_copy(
            imgs_ref.at[pl.ds(ibase + o, _PACK_W)], lo_v.at[slot], lo_sem)
        hi_d = pltpu.async_copy(
            imgs_ref.at[pl.ds(ibase + _NWORDS + o, _PACK_W)],
            hi_v.at[slot], hi_sem)
        return lo_d, hi_d

    pend = {0: _start_pack(0, 0)}
    for s in range(_PACK_CHUNKS):
        slot = s % 2
        if s + 1 < _PACK_CHUNKS:
            pend[s + 1] = _start_pack(s + 1, (s + 1) % 2)
        lo_d, hi_d = pend.pop(s)
        lo_d.wait()
        hi_d.wait()
        o = s * _PACK_W

        @plsc.parallel_loop(0, _PACK_W // 16, unroll=4)
        def _pack(g):
            lo = plsc.bitcast(lo_v[slot, pl.ds(g * 16, 16)], jnp.int32)
            hi = plsc.bitcast(hi_v[slot, pl.ds(g * 16, 16)], jnp.int32)
            lor = lax.shift_right_logical(lo + 0x8000, 16)
            hir = lax.shift_right_logical(hi + 0x8000, 16)
            img_v[pl.ds(o + g * 16, 16)] = lor | (hir << 16)

    lane = lax.iota(jnp.int32, 16)
    lanef = lane.astype(jnp.float32)

    def _start_dvf(ch, slot):
        crow = row0 + ch * _CHUNK_ROWS
        dsrc = (b * _H + crow) * 2 * _W
        return pltpu.async_copy(
            dvf_ref.at[pl.ds(dsrc, _CHUNK_PIX * 2)], dvf_v.at[slot], dvf_sem)

    dvf_pend = {0: _start_dvf(0, 0)}
    out_pend = {}
    for ch in range(_NCHUNKS):
        slot = ch % 2
        if ch + 1 < _NCHUNKS:
            dvf_pend[ch + 1] = _start_dvf(ch + 1, (ch + 1) % 2)
        dvf_pend.pop(ch).wait()
        if ch >= 2:
            out_pend.pop(ch - 2).wait()
        crow = row0 + ch * _CHUNK_ROWS

        def _row(r, carry):
            rowf = (crow + r).astype(jnp.float32)

            @plsc.parallel_loop(0, _GROUPS_PER_ROW, unroll=4)
            def _grp(t):
                p0 = r * _W + t * 16
                doff = 2 * r * _W + t * 16
                dx = dvf_v[slot, pl.ds(doff, 16)]
                dy = dvf_v[slot, pl.ds(doff + _W, 16)]

                fx = (t * 16).astype(jnp.float32) + lanef + dx
                fy = rowf + dy
                x0 = fx.astype(jnp.int32)
                y0 = fy.astype(jnp.int32)
                x1 = x0 + 1
                y1 = y0 + 1
                x0 = jnp.clip(x0, 0, _W - 1)
                x1 = jnp.clip(x1, 0, _W - 1)
                y0 = jnp.clip(y0, 0, _H - 1)
                y1 = jnp.clip(y1, 0, _H - 1)

                ry0 = y0 * _W
                ry1 = y1 * _W
                m0 = y0 < (_H // 2)
                m1 = y1 < (_H // 2)
                off0 = jnp.where(m0, ry0, ry0 - _NWORDS)
                off1 = jnp.where(m1, ry1, ry1 - _NWORDS)
                sh0 = jnp.where(m0, 16, 0)
                sh1 = jnp.where(m1, 16, 0)

                wa = plsc.load_gather(img_v, [off0 + x0])
                wb = plsc.load_gather(img_v, [off1 + x0])
                wc = plsc.load_gather(img_v, [off0 + x1])
                wd = plsc.load_gather(img_v, [off1 + x1])
                va = plsc.bitcast(wa << sh0, jnp.float32)
                vb = plsc.bitcast(wb << sh1, jnp.float32)
                vc = plsc.bitcast(wc << sh0, jnp.float32)
                vd = plsc.bitcast(wd << sh1, jnp.float32)

                x0f = x0.astype(jnp.float32)
                x1f = x1.astype(jnp.float32)
                y0f = y0.astype(jnp.float32)
                y1f = y1.astype(jnp.float32)
                wx1 = x1f - fx
                wx0 = fx - x0f
                wy1 = y1f - fy
                wy0 = fy - y0f
                res = ((wx1 * wy1) * va + (wx1 * wy0) * vb
                       + (wx0 * wy1) * vc + (wx0 * wy0) * vd)
                out_v[slot, pl.ds(p0, 16)] = res

            return carry

        lax.fori_loop(0, _CHUNK_ROWS, _row, jnp.int32(0))

        dst = ibase + crow * _W
        out_pend[ch] = pltpu.async_copy(
            out_v.at[slot], out_ref.at[pl.ds(dst, _CHUNK_PIX)], out_sem)
    for ch in sorted(out_pend):
        out_pend[ch].wait()


@jax.jit
def _run(imgs_flat, dvf_flat):
    mesh = plsc.VectorSubcoreMesh(core_axis_name="c", subcore_axis_name="s")
    fn = pl.kernel(
        _sc_body,
        out_type=jax.ShapeDtypeStruct((_B * _NPIX,), jnp.float32),
        name="bilerp_sc",
        mesh=mesh,
        scratch_types=[
            pltpu.VMEM((_NWORDS,), jnp.int32),
            pltpu.VMEM((2, _PACK_W), jnp.float32),
            pltpu.VMEM((2, _PACK_W), jnp.float32),
            pltpu.VMEM((2, _CHUNK_PIX * 2), jnp.float32),
            pltpu.VMEM((2, _CHUNK_PIX), jnp.float32),
            pltpu.SemaphoreType.DMA,
            pltpu.SemaphoreType.DMA,
            pltpu.SemaphoreType.DMA,
            pltpu.SemaphoreType.DMA,
        ],
        compiler_params=pltpu.CompilerParams(needs_layout_passes=False),
    )
    return fn(imgs_flat, dvf_flat)


def kernel(imgs, dvfs):
    B, H, W, C = imgs.shape
    imgs_flat = imgs.reshape(-1)
    dvf_flat = jnp.transpose(dvfs, (0, 1, 3, 2)).reshape(-1)
    out = _run(imgs_flat, dvf_flat)
    return out.reshape(B, H, W, C)


# speedup vs baseline: 1.1619x; 1.0288x over previous
import functools

import jax
import jax.numpy as jnp
from jax import lax
from jax.experimental import pallas as pl
from jax.experimental.pallas import tpu as pltpu
from jax.experimental.pallas import tpu_sc as plsc

_B, _H, _W = 8, 384, 384
_NPIX = _H * _W
_NWORDS = _NPIX // 2
_TILES_PER_IMG = 4
_ROWS_PER_TILE = _H // _TILES_PER_IMG
_CHUNK_ROWS = 16
_CHUNK_PIX = _CHUNK_ROWS * _W
_NCHUNKS = _ROWS_PER_TILE // _CHUNK_ROWS
_GROUPS_PER_ROW = _W // 16
_PACK_CHUNKS = 16
_PACK_W = _NWORDS // _PACK_CHUNKS


def _sc_body(imgs_ref, dvf_ref, out_ref,
             img_v, shared_v, lo_v, hi_v, dvf_v, out_v,
             lo_sem, hi_sem, dvf_sem, out_sem):
    cid = lax.axis_index("c")
    sid = lax.axis_index("s")
    b = cid * 4 + sid // _TILES_PER_IMG
    bslot = sid // _TILES_PER_IMG
    q = sid % _TILES_PER_IMG
    row0 = q * _ROWS_PER_TILE
    ibase = b * _NPIX

    qwords = _NWORDS // _TILES_PER_IMG
    qbase = q * qwords
    nq = qwords // _PACK_W

    def _start_pack(s, slot):
        o = qbase + s * _PACK_W
        lo_d = pltpu.async_copy(
            imgs_ref.at[pl.ds(ibase + o, _PACK_W)], lo_v.at[slot], lo_sem)
        hi_d = pltpu.async_copy(
            imgs_ref.at[pl.ds(ibase + _NWORDS + o, _PACK_W)],
            hi_v.at[slot], hi_sem)
        return lo_d, hi_d

    pend = {0: _start_pack(0, 0)}
    for s in range(nq):
        slot = s % 2
        if s + 1 < nq:
            pend[s + 1] = _start_pack(s + 1, (s + 1) % 2)
        lo_d, hi_d = pend.pop(s)
        lo_d.wait()
        hi_d.wait()
        o = qbase + s * _PACK_W

        @plsc.parallel_loop(0, _PACK_W // 16, unroll=4)
        def _pack(g):
            lo = plsc.bitcast(lo_v[slot, pl.ds(g * 16, 16)], jnp.int32)
            hi = plsc.bitcast(hi_v[slot, pl.ds(g * 16, 16)], jnp.int32)
            lor = lax.shift_right_logical(lo + 0x8000, 16)
            hir = lax.shift_right_logical(hi + 0x8000, 16)
            img_v[pl.ds(o + g * 16, 16)] = lor | (hir << 16)

    pw = qwords // 3
    for p in range(3 * _TILES_PER_IMG):
        qq, sub = p // 3, p % 3

        @pl.when(q == qq)
        def _publish():
            pltpu.sync_copy(img_v.at[pl.ds(qbase + sub * pw, pw)],
                            shared_v.at[pl.ds(bslot * pw, pw)])
        plsc.subcore_barrier()

        @pl.when(q != qq)
        def _fetch():
            pltpu.sync_copy(shared_v.at[pl.ds(bslot * pw, pw)],
                            img_v.at[pl.ds(p * pw, pw)])
        plsc.subcore_barrier()

    lane = lax.iota(jnp.int32, 16)
    lanef = lane.astype(jnp.float32)

    def _start_dvf(ch, slot):
        crow = row0 + ch * _CHUNK_ROWS
        dsrc = (b * _H + crow) * 2 * _W
        return pltpu.async_copy(
            dvf_ref.at[pl.ds(dsrc, _CHUNK_PIX * 2)], dvf_v.at[slot], dvf_sem)

    dvf_pend = {0: _start_dvf(0, 0)}
    out_pend = {}
    for ch in range(_NCHUNKS):
        slot = ch % 2
        if ch + 1 < _NCHUNKS:
            dvf_pend[ch + 1] = _start_dvf(ch + 1, (ch + 1) % 2)
        dvf_pend.pop(ch).wait()
        if ch >= 2:
            out_pend.pop(ch - 2).wait()
        crow = row0 + ch * _CHUNK_ROWS

        def _row(r, carry):
            rowf = (crow + r).astype(jnp.float32)

            @plsc.parallel_loop(0, _GROUPS_PER_ROW, unroll=4)
            def _grp(t):
                p0 = r * _W + t * 16
                doff = 2 * r * _W + t * 16
                dx = dvf_v[slot, pl.ds(doff, 16)]
                dy = dvf_v[slot, pl.ds(doff + _W, 16)]

                fx = (t * 16).astype(jnp.float32) + lanef + dx
                fy = rowf + dy
                x0 = fx.astype(jnp.int32)
                y0 = fy.astype(jnp.int32)
                x1 = x0 + 1
                y1 = y0 + 1
                x0 = jnp.clip(x0, 0, _W - 1)
                x1 = jnp.clip(x1, 0, _W - 1)
                y0 = jnp.clip(y0, 0, _H - 1)
                y1 = jnp.clip(y1, 0, _H - 1)

                ry0 = y0 * _W
                ry1 = y1 * _W
                m0 = y0 < (_H // 2)
                m1 = y1 < (_H // 2)
                off0 = jnp.where(m0, ry0, ry0 - _NWORDS)
                off1 = jnp.where(m1, ry1, ry1 - _NWORDS)
                sh0 = jnp.where(m0, 16, 0)
                sh1 = jnp.where(m1, 16, 0)

                wa = plsc.load_gather(img_v, [off0 + x0])
                wb = plsc.load_gather(img_v, [off1 + x0])
                wc = plsc.load_gather(img_v, [off0 + x1])
                wd = plsc.load_gather(img_v, [off1 + x1])
                va = plsc.bitcast(wa << sh0, jnp.float32)
                vb = plsc.bitcast(wb << sh1, jnp.float32)
                vc = plsc.bitcast(wc << sh0, jnp.float32)
                vd = plsc.bitcast(wd << sh1, jnp.float32)

                x0f = x0.astype(jnp.float32)
                x1f = x1.astype(jnp.float32)
                y0f = y0.astype(jnp.float32)
                y1f = y1.astype(jnp.float32)
                wx1 = x1f - fx
                wx0 = fx - x0f
                wy1 = y1f - fy
                wy0 = fy - y0f
                res = ((wx1 * wy1) * va + (wx1 * wy0) * vb
                       + (wx0 * wy1) * vc + (wx0 * wy0) * vd)
                out_v[slot, pl.ds(p0, 16)] = res

            return carry

        lax.fori_loop(0, _CHUNK_ROWS, _row, jnp.int32(0))

        dst = ibase + crow * _W
        out_pend[ch] = pltpu.async_copy(
            out_v.at[slot], out_ref.at[pl.ds(dst, _CHUNK_PIX)], out_sem)
    for ch in sorted(out_pend):
        out_pend[ch].wait()


@jax.jit
def _run(imgs_flat, dvf_flat):
    mesh = plsc.VectorSubcoreMesh(core_axis_name="c", subcore_axis_name="s")
    fn = pl.kernel(
        _sc_body,
        out_type=jax.ShapeDtypeStruct((_B * _NPIX,), jnp.float32),
        name="bilerp_sc",
        mesh=mesh,
        scratch_types=[
            pltpu.VMEM((_NWORDS,), jnp.int32),
            pltpu.VMEM_SHARED((_NWORDS // 3,), jnp.int32),
            pltpu.VMEM((2, _PACK_W), jnp.float32),
            pltpu.VMEM((2, _PACK_W), jnp.float32),
            pltpu.VMEM((2, _CHUNK_PIX * 2), jnp.float32),
            pltpu.VMEM((2, _CHUNK_PIX), jnp.float32),
            pltpu.SemaphoreType.DMA,
            pltpu.SemaphoreType.DMA,
            pltpu.SemaphoreType.DMA,
            pltpu.SemaphoreType.DMA,
        ],
        compiler_params=pltpu.CompilerParams(needs_layout_passes=False),
    )
    return fn(imgs_flat, dvf_flat)


def kernel(imgs, dvfs):
    B, H, W, C = imgs.shape
    imgs_flat = imgs.reshape(-1)
    dvf_flat = jnp.transpose(dvfs, (0, 1, 3, 2)).reshape(-1)
    out = _run(imgs_flat, dvf_flat)
    return out.reshape(B, H, W, C)


# speedup vs baseline: 1.1750x; 1.0113x over previous
import functools

import jax
import jax.numpy as jnp
from jax import lax
from jax.experimental import pallas as pl
from jax.experimental.pallas import tpu as pltpu
from jax.experimental.pallas import tpu_sc as plsc

_B, _H, _W = 8, 384, 384
_NPIX = _H * _W
_NWORDS = _NPIX // 2
_TILES_PER_IMG = 4
_ROWS_PER_TILE = _H // _TILES_PER_IMG
_CHUNK_ROWS = 16
_CHUNK_PIX = _CHUNK_ROWS * _W
_NCHUNKS = _ROWS_PER_TILE // _CHUNK_ROWS
_GROUPS_PER_ROW = _W // 16
_PACK_CHUNKS = 16
_PACK_W = _NWORDS // _PACK_CHUNKS


def _sc_body(imgs_ref, dvf_ref, out_ref,
             img_v, shared_v, lo_v, hi_v, dvf_v, out_v,
             lo_sem, hi_sem, dvf_sem, out_sem):
    cid = lax.axis_index("c")
    sid = lax.axis_index("s")
    b = cid * 4 + sid // _TILES_PER_IMG
    bslot = sid // _TILES_PER_IMG
    q = sid % _TILES_PER_IMG
    row0 = q * _ROWS_PER_TILE
    ibase = b * _NPIX

    qwords = _NWORDS // _TILES_PER_IMG
    qbase = q * qwords
    nq = qwords // _PACK_W

    def _start_pack(s, slot):
        o = qbase + s * _PACK_W
        lo_d = pltpu.async_copy(
            imgs_ref.at[pl.ds(ibase + o, _PACK_W)], lo_v.at[slot], lo_sem)
        hi_d = pltpu.async_copy(
            imgs_ref.at[pl.ds(ibase + _NWORDS + o, _PACK_W)],
            hi_v.at[slot], hi_sem)
        return lo_d, hi_d

    pend = {0: _start_pack(0, 0)}
    for s in range(nq):
        slot = s % 2
        if s + 1 < nq:
            pend[s + 1] = _start_pack(s + 1, (s + 1) % 2)
        lo_d, hi_d = pend.pop(s)
        lo_d.wait()
        hi_d.wait()
        o = qbase + s * _PACK_W

        @plsc.parallel_loop(0, _PACK_W // 16, unroll=4)
        def _pack(g):
            lo = plsc.bitcast(lo_v[slot, pl.ds(g * 16, 16)], jnp.int32)
            hi = plsc.bitcast(hi_v[slot, pl.ds(g * 16, 16)], jnp.int32)
            lor = lax.shift_right_logical(lo + 0x8000, 16)
            hir = lax.shift_right_logical(hi + 0x8000, 16)
            img_v[pl.ds(o + g * 16, 16)] = lor | (hir << 16)

    pw = qwords // 3
    for p in range(3 * _TILES_PER_IMG):
        qq, sub = p // 3, p % 3

        @pl.when(q == qq)
        def _publish():
            pltpu.sync_copy(img_v.at[pl.ds(qbase + sub * pw, pw)],
                            shared_v.at[pl.ds(bslot * pw, pw)])
        plsc.subcore_barrier()

        @pl.when(q != qq)
        def _fetch():
            pltpu.sync_copy(shared_v.at[pl.ds(bslot * pw, pw)],
                            img_v.at[pl.ds(p * pw, pw)])
        plsc.subcore_barrier()

    lane = lax.iota(jnp.int32, 16)
    lanef = lane.astype(jnp.float32)

    def _start_dvf(ch, slot):
        crow = row0 + ch * _CHUNK_ROWS
        dsrc = (b * _H + crow) * 2 * _W
        return pltpu.async_copy(
            dvf_ref.at[pl.ds(dsrc, _CHUNK_PIX * 2)], dvf_v.at[slot], dvf_sem)

    dvf_pend = {0: _start_dvf(0, 0)}
    out_pend = {}
    for ch in range(_NCHUNKS):
        slot = ch % 2
        if ch + 1 < _NCHUNKS:
            dvf_pend[ch + 1] = _start_dvf(ch + 1, (ch + 1) % 2)
        dvf_pend.pop(ch).wait()
        if ch >= 2:
            out_pend.pop(ch - 2).wait()
        crow = row0 + ch * _CHUNK_ROWS

        def _row(r, carry):
            rowf = (crow + r).astype(jnp.float32)

            @plsc.parallel_loop(0, _GROUPS_PER_ROW, unroll=4)
            def _grp(t):
                p0 = r * _W + t * 16
                doff = 2 * r * _W + t * 16
                dx = dvf_v[slot, pl.ds(doff, 16)]
                dy = dvf_v[slot, pl.ds(doff + _W, 16)]

                fx = (t * 16).astype(jnp.float32) + lanef + dx
                fy = rowf + dy
                x0 = fx.astype(jnp.int32)
                y0 = fy.astype(jnp.int32)
                x1 = x0 + 1
                y1 = y0 + 1
                x0 = jnp.clip(x0, 0, _W - 1)
                x1 = jnp.clip(x1, 0, _W - 1)
                y0 = jnp.clip(y0, 0, _H - 1)
                y1 = jnp.clip(y1, 0, _H - 1)

                ry0 = y0 * _W
                ry1 = y1 * _W
                m0 = y0 < (_H // 2)
                m1 = y1 < (_H // 2)
                off0 = jnp.where(m0, ry0, ry0 - _NWORDS)
                off1 = jnp.where(m1, ry1, ry1 - _NWORDS)
                sh0 = jnp.where(m0, 16, 0)
                sh1 = jnp.where(m1, 16, 0)

                wa = plsc.load_gather(img_v, [off0 + x0])
                wb = plsc.load_gather(img_v, [off1 + x0])
                wc = plsc.load_gather(img_v, [off0 + x1])
                wd = plsc.load_gather(img_v, [off1 + x1])
                va = plsc.bitcast(wa << sh0, jnp.float32)
                vb = plsc.bitcast(wb << sh1, jnp.float32)
                vc = plsc.bitcast(wc << sh0, jnp.float32)
                vd = plsc.bitcast(wd << sh1, jnp.float32)

                x0f = x0.astype(jnp.float32)
                x1f = x1.astype(jnp.float32)
                y0f = y0.astype(jnp.float32)
                y1f = y1.astype(jnp.float32)
                wx1 = x1f - fx
                wx0 = fx - x0f
                wy1 = y1f - fy
                wy0 = fy - y0f
                res = (wy1 * (wx1 * va + wx0 * vc)
                       + wy0 * (wx1 * vb + wx0 * vd))
                out_v[slot, pl.ds(p0, 16)] = res

            return carry

        lax.fori_loop(0, _CHUNK_ROWS, _row, jnp.int32(0))

        dst = ibase + crow * _W
        out_pend[ch] = pltpu.async_copy(
            out_v.at[slot], out_ref.at[pl.ds(dst, _CHUNK_PIX)], out_sem)
    for ch in sorted(out_pend):
        out_pend[ch].wait()


@jax.jit
def _run(imgs_flat, dvf_flat):
    mesh = plsc.VectorSubcoreMesh(core_axis_name="c", subcore_axis_name="s")
    fn = pl.kernel(
        _sc_body,
        out_type=jax.ShapeDtypeStruct((_B * _NPIX,), jnp.float32),
        name="bilerp_sc",
        mesh=mesh,
        scratch_types=[
            pltpu.VMEM((_NWORDS,), jnp.int32),
            pltpu.VMEM_SHARED((_NWORDS // 3,), jnp.int32),
            pltpu.VMEM((2, _PACK_W), jnp.float32),
            pltpu.VMEM((2, _PACK_W), jnp.float32),
            pltpu.VMEM((2, _CHUNK_PIX * 2), jnp.float32),
            pltpu.VMEM((2, _CHUNK_PIX), jnp.float32),
            pltpu.SemaphoreType.DMA,
            pltpu.SemaphoreType.DMA,
            pltpu.SemaphoreType.DMA,
            pltpu.SemaphoreType.DMA,
        ],
        compiler_params=pltpu.CompilerParams(needs_layout_passes=False),
    )
    return fn(imgs_flat, dvf_flat)


def kernel(imgs, dvfs):
    B, H, W, C = imgs.shape
    imgs_flat = imgs.reshape(-1)
    dvf_flat = jnp.transpose(dvfs, (0, 1, 3, 2)).reshape(-1)
    out = _run(imgs_flat, dvf_flat)
    return out.reshape(B, H, W, C)
